# baseline (speedup 1.0000x reference)
import functools

import jax
import jax.numpy as jnp
from jax import lax
from jax.experimental import pallas as pl
from jax.experimental.pallas import tpu as pltpu
from jax.experimental.pallas import tpu_sc as plsc

_B = 4096
_L = 200
_D = 64
_FC = 26
_VC = 100000
_VS = 1000000
_K = 32
_NC = 2
_NS = 16
_NW = _NC * _NS
_CH = 128

_ATT_ROWS = _B * _L
_ATT_PER_W = _ATT_ROWS // _NW
_ATT_CHUNKS = _ATT_PER_W // _CH
_CAT_ROWS = _B * _FC
_CAT_PER_W = _CAT_ROWS // _NW
_CAT_CHUNKS = _CAT_PER_W // _CH
_REP_ROWS = _B * _K
_REP_PER_W = _REP_ROWS // _NW
_REP_CHUNKS = _REP_PER_W // _CH

_TCOL = 4096
_TGRID = (_VS + _TCOL - 1) // _TCOL


def _tx_body(tt_ref, out_ref):
    t = jnp.transpose(tt_ref[...], (1, 0))
    out_ref[:, :_D] = t
    out_ref[:, _D:] = t


def _tx(table):
    tt = jnp.transpose(table)
    return pl.pallas_call(
        _tx_body,
        grid=(_TGRID,),
        in_specs=[pl.BlockSpec((_D, _TCOL), lambda i: (0, i))],
        out_specs=pl.BlockSpec((_TCOL, 2 * _D), lambda i: (i, 0)),
        out_shape=jax.ShapeDtypeStruct((_VS, 2 * _D), jnp.float32),
    )(tt)


def _sc_att_body(seq_hbm, att2, att_out, idx0, idx1, rows0, rows1,
                 sem0, sem1):
    wid = lax.axis_index("s") * _NC + lax.axis_index("c")
    base0 = wid * _ATT_PER_W

    pltpu.sync_copy(seq_hbm.at[pl.ds(base0, _CH)], idx0)
    pltpu.async_copy(att2.at[idx0], rows0, sem0)
    pltpu.sync_copy(seq_hbm.at[pl.ds(base0 + _CH, _CH)], idx1)
    pltpu.async_copy(att2.at[idx1], rows1, sem1)

    def pair(jj, carry):
        b0 = base0 + (2 * jj) * _CH
        pltpu.make_async_copy(att2.at[idx0], rows0, sem0).wait()
        pltpu.sync_copy(rows0, att_out.at[pl.ds(b0, _CH)])

        @pl.when(jj < _ATT_CHUNKS // 2 - 1)
        def _():
            pltpu.sync_copy(seq_hbm.at[pl.ds(b0 + 2 * _CH, _CH)], idx0)
            pltpu.async_copy(att2.at[idx0], rows0, sem0)

        pltpu.make_async_copy(att2.at[idx1], rows1, sem1).wait()
        pltpu.sync_copy(rows1, att_out.at[pl.ds(b0 + _CH, _CH)])

        @pl.when(jj < _ATT_CHUNKS // 2 - 1)
        def _():
            pltpu.sync_copy(seq_hbm.at[pl.ds(b0 + 3 * _CH, _CH)], idx1)
            pltpu.async_copy(att2.at[idx1], rows1, sem1)

        return carry

    lax.fori_loop(0, _ATT_CHUNKS // 2, pair, 0)


def _sc_rep_body(g_hbm, rep2, rep_out, idx_v, rows_v, sem):
    wid = lax.axis_index("s") * _NC + lax.axis_index("c")
    base0 = wid * _REP_PER_W

    def chunk(j, carry):
        base = base0 + j * _CH
        pltpu.sync_copy(g_hbm.at[pl.ds(base, _CH)], idx_v)
        pltpu.async_copy(rep2.at[idx_v], rows_v, sem).wait()
        pltpu.sync_copy(rows_v, rep_out.at[pl.ds(base, _CH)])
        return carry

    lax.fori_loop(0, _REP_CHUNKS, chunk, 0)


_BAT = 16


def _sc_cat_body(catt_hbm, cat_tbl, cat_out, idx_v, rows_v, sem):
    wid = lax.axis_index("s") * _NC + lax.axis_index("c")
    base0 = wid * _CAT_PER_W

    def chunk(j, carry):
        base = base0 + j * _CH
        field = base // _B
        pltpu.sync_copy(catt_hbm.at[pl.ds(base, _CH)], idx_v)

        def batch(jj, c2):
            i0 = jj * _BAT
            rv = idx_v[pl.ds(i0, _BAT)]
            for t in range(_BAT):
                pltpu.async_copy(cat_tbl.at[field, rv[t]],
                                 rows_v.at[i0 + t], sem)
            for t in range(_BAT):
                pltpu.make_async_copy(cat_tbl.at[field, rv[t]],
                                      rows_v.at[i0 + t], sem).wait()
            return c2

        lax.fori_loop(0, _CH // _BAT, batch, 0)
        pltpu.sync_copy(rows_v, cat_out.at[pl.ds(base, _CH)])
        return carry

    lax.fori_loop(0, _CAT_CHUNKS, chunk, 0)


@functools.lru_cache(maxsize=None)
def _sc_kernels():
    mesh = plsc.VectorSubcoreMesh(core_axis_name="c", subcore_axis_name="s",
                                  num_cores=_NC, num_subcores=_NS)
    wide = [
        pltpu.VMEM((_CH,), jnp.int32),
        pltpu.VMEM((_CH, 2 * _D), jnp.float32),
        pltpu.SemaphoreType.DMA,
    ]
    narrow = [
        pltpu.VMEM((_CH,), jnp.int32),
        pltpu.VMEM((_CH, _D), jnp.float32),
        pltpu.SemaphoreType.DMA,
    ]
    katt = pl.kernel(
        _sc_att_body,
        out_type=jax.ShapeDtypeStruct((_ATT_ROWS, 2 * _D), jnp.float32),
        mesh=mesh,
        scratch_types=[
            pltpu.VMEM((_CH,), jnp.int32),
            pltpu.VMEM((_CH,), jnp.int32),
            pltpu.VMEM((_CH, 2 * _D), jnp.float32),
            pltpu.VMEM((_CH, 2 * _D), jnp.float32),
            pltpu.SemaphoreType.DMA,
            pltpu.SemaphoreType.DMA,
        ],
    )
    kcat = pl.kernel(
        _sc_cat_body,
        out_type=jax.ShapeDtypeStruct((_CAT_ROWS, _D), jnp.float32),
        mesh=mesh,
        scratch_types=narrow,
    )
    krep = pl.kernel(
        _sc_rep_body,
        out_type=jax.ShapeDtypeStruct((_REP_ROWS, 2 * _D), jnp.float32),
        mesh=mesh,
        scratch_types=wide,
    )
    return katt, kcat, krep


_BB1 = 128


def _tc1_body(att_ref, cat_ref, seq_ref, xnum_ref, xmask_ref, nw1_ref, np_ref,
              mw1_ref, mp_ref, cproj_ref, fc1r_ref, fc1b_ref,
              part_ref, w_ref, g_ref):
    f32 = jnp.float32
    cats = []
    for f in range(_FC):
        ce = jnp.dot(cat_ref[f], cproj_ref[f], preferred_element_type=f32)
        cats.append(ce)
    q = cats[0]

    nm = jnp.dot(jnp.dot(xnum_ref[...], nw1_ref[...],
                         preferred_element_type=f32), np_ref[...],
                 preferred_element_type=f32) * (1.0 / 13.0)
    mm = jnp.dot(jnp.dot(xmask_ref[...], mw1_ref[...],
                         preferred_element_type=f32), mp_ref[...],
                 preferred_element_type=f32) * (1.0 / 13.0)

    att = att_ref[...]
    qext = jnp.concatenate([q, jnp.zeros((_BB1, _D), f32)], axis=1)
    s = jnp.sum(att * qext[:, None, :], axis=-1) * 0.125
    iota_l = lax.broadcasted_iota(jnp.int32, (_BB1, _L), 1)
    bias = (jnp.float32(_L - 1) - iota_l.astype(f32)) * (1.0 / 50.0)
    s = s - bias
    seqb = seq_ref[...]
    s = jnp.where(seqb == 0, jnp.float32(-1e9), s)

    vals, gsel = [], []
    for _ in range(_K):
        m = jnp.max(s, axis=1, keepdims=True)
        cand = jnp.where(s == m, iota_l, jnp.int32(_L))
        ji = jnp.min(cand, axis=1, keepdims=True)
        onehot = iota_l == ji
        gk = jnp.sum(jnp.where(onehot, seqb, 0), axis=1, keepdims=True)
        s = jnp.where(onehot, jnp.float32(-3e38), s)
        vals.append(m)
        gsel.append(gk)
    vals = jnp.concatenate(vals, axis=1)
    gsel = jnp.concatenate(gsel, axis=1)
    vmax = jnp.max(vals, axis=1, keepdims=True)
    e = jnp.exp(vals - vmax)
    w = e / jnp.sum(e, axis=1, keepdims=True)

    feats = jnp.concatenate([nm, mm] + cats, axis=1)
    part = jnp.dot(feats, fc1r_ref[...], preferred_element_type=f32)
    part = part + fc1b_ref[...]

    part_ref[...] = part
    w_ref[...] = w
    g_ref[...] = gsel


_BB2 = 256


def _tc2_body(rep_ref, w_ref, part_ref, fc1u_ref, auxw_ref, auxb_ref,
              fc2w_ref, fc2b_ref, logit_ref, prob_ref, aux_ref):
    f32 = jnp.float32
    w = w_ref[...]
    u = jnp.zeros((_BB2, _D), f32)
    for k in range(_K):
        u = u + w[:, k:k + 1] * rep_ref[:, k * 2 * _D:k * 2 * _D + _D]
    aux = jnp.dot(u, auxw_ref[...], preferred_element_type=f32) + auxb_ref[...]
    h = jnp.maximum(part_ref[...] +
                    jnp.dot(u, fc1u_ref[...], preferred_element_type=f32), 0.0)
    logit = jnp.dot(h, fc2w_ref[...], preferred_element_type=f32) + fc2b_ref[...]
    logit_ref[...] = logit
    prob_ref[...] = jax.nn.sigmoid(logit)
    aux_ref[...] = aux


def _tc1(att_g, cat_raw, seq, xnum, xmask, nw1, npj, mw1, mpj, cproj,
         fc1r, fc1b):
    grid = (_B // _BB1,)
    return pl.pallas_call(
        _tc1_body,
        grid=grid,
        in_specs=[
            pl.BlockSpec((_BB1, _L, 2 * _D), lambda i: (i, 0, 0)),
            pl.BlockSpec((_FC, _BB1, _D), lambda i: (0, i, 0)),
            pl.BlockSpec((_BB1, _L), lambda i: (i, 0)),
            pl.BlockSpec((_BB1, 13), lambda i: (i, 0)),
            pl.BlockSpec((_BB1, 13), lambda i: (i, 0)),
            pl.BlockSpec((13, 16), lambda i: (0, 0)),
            pl.BlockSpec((16, _D), lambda i: (0, 0)),
            pl.BlockSpec((13, 16), lambda i: (0, 0)),
            pl.BlockSpec((16, _D), lambda i: (0, 0)),
            pl.BlockSpec((_FC, _D, _D), lambda i: (0, 0, 0)),
            pl.BlockSpec((28 * _D, 512), lambda i: (0, 0)),
            pl.BlockSpec((1, 512), lambda i: (0, 0)),
        ],
        out_specs=[
            pl.BlockSpec((_BB1, 512), lambda i: (i, 0)),
            pl.BlockSpec((_BB1, _K), lambda i: (i, 0)),
            pl.BlockSpec((_BB1, _K), lambda i: (i, 0)),
        ],
        out_shape=[
            jax.ShapeDtypeStruct((_B, 512), jnp.float32),
            jax.ShapeDtypeStruct((_B, _K), jnp.float32),
            jax.ShapeDtypeStruct((_B, _K), jnp.int32),
        ],
    )(att_g, cat_raw, seq, xnum, xmask, nw1, npj, mw1, mpj, cproj, fc1r, fc1b)


def _tc2(rep2, w, part, fc1u, auxw, auxb, fc2w, fc2b):
    grid = (_B // _BB2,)
    return pl.pallas_call(
        _tc2_body,
        grid=grid,
        in_specs=[
            pl.BlockSpec((_BB2, _K * 2 * _D), lambda i: (i, 0)),
            pl.BlockSpec((_BB2, _K), lambda i: (i, 0)),
            pl.BlockSpec((_BB2, 512), lambda i: (i, 0)),
            pl.BlockSpec((_D, 512), lambda i: (0, 0)),
            pl.BlockSpec((_D, 1), lambda i: (0, 0)),
            pl.BlockSpec((1, 1), lambda i: (0, 0)),
            pl.BlockSpec((512, 1), lambda i: (0, 0)),
            pl.BlockSpec((1, 1), lambda i: (0, 0)),
        ],
        out_specs=[
            pl.BlockSpec((_BB2, 1), lambda i: (i, 0)),
            pl.BlockSpec((_BB2, 1), lambda i: (i, 0)),
            pl.BlockSpec((_BB2, 1), lambda i: (i, 0)),
        ],
        out_shape=[
            jax.ShapeDtypeStruct((_B, 1), jnp.float32),
            jax.ShapeDtypeStruct((_B, 1), jnp.float32),
            jax.ShapeDtypeStruct((_B, 1), jnp.float32),
        ],
    )(rep2, w, part, fc1u, auxw, auxb, fc2w, fc2b)


def kernel(X_num, X_mask, X_cat, seq, num_W1, num_P, mask_W1, mask_P,
           cat_tables, cat_proj, att_table, rep_table, aux_w, aux_b,
           fc1_w, fc1_b, fc2_w, fc2_b):
    seq = seq.astype(jnp.int32)
    X_cat = X_cat.astype(jnp.int32)
    seq_flat = seq.reshape(-1)
    catt = X_cat.T.reshape(-1)

    katt, kcat, krep = _sc_kernels()

    cat_raw = kcat(catt, cat_tables)
    att2 = _tx(att_table)
    att_g = katt(seq_flat, att2)

    part, w, g = _tc1(
        att_g.reshape(_B, _L, 2 * _D), cat_raw.reshape(_FC, _B, _D), seq,
        X_num, X_mask, num_W1, num_P, mask_W1, mask_P, cat_proj,
        fc1_w[_D:], fc1_b.reshape(1, 512))

    rep2 = _tx(rep_table)
    rep_g = krep(g.reshape(-1), rep2)

    logit, prob, aux = _tc2(
        rep_g.reshape(_B, _K * 2 * _D), w, part, fc1_w[:_D],
        aux_w.reshape(_D, 1), aux_b.reshape(1, 1), fc2_w,
        fc2_b.reshape(1, 1))

    return (logit[:, 0], prob[:, 0], aux[:, 0])


# speedup vs baseline: 1.0174x; 1.0174x over previous
import functools

import jax
import jax.numpy as jnp
from jax import lax
from jax.experimental import pallas as pl
from jax.experimental.pallas import tpu as pltpu
from jax.experimental.pallas import tpu_sc as plsc

_B = 4096
_L = 200
_D = 64
_FC = 26
_VC = 100000
_VS = 1000000
_K = 32
_NC = 2
_NS = 16
_NW = _NC * _NS
_CH = 128

_ATT_ROWS = _B * _L
_ATT_PER_W = _ATT_ROWS // _NW
_ATT_CHUNKS = _ATT_PER_W // _CH
_CAT_ROWS = _B * _FC
_CAT_PER_W = _CAT_ROWS // _NW
_CAT_CHUNKS = _CAT_PER_W // _CH
_REP_ROWS = _B * _K
_REP_PER_W = _REP_ROWS // _NW
_REP_CHUNKS = _REP_PER_W // _CH

_TCOL = 4096
_TGRID = (_VS + _TCOL - 1) // _TCOL


def _tx_body(tt_ref, out_ref):
    t = jnp.transpose(tt_ref[...], (1, 0))
    out_ref[:, :_D] = t
    out_ref[:, _D:] = t


def _tx(table):
    tt = jnp.transpose(table)
    return pl.pallas_call(
        _tx_body,
        grid=(_TGRID,),
        in_specs=[pl.BlockSpec((_D, _TCOL), lambda i: (0, i))],
        out_specs=pl.BlockSpec((_TCOL, 2 * _D), lambda i: (i, 0)),
        out_shape=jax.ShapeDtypeStruct((_VS, 2 * _D), jnp.float32),
    )(tt)


def _sc_att_body(seq_hbm, att2, att_out, idx_v, rows_v, sem):
    wid = lax.axis_index("s") * _NC + lax.axis_index("c")
    base0 = wid * _ATT_PER_W

    def chunk(j, carry):
        base = base0 + j * _CH
        pltpu.sync_copy(seq_hbm.at[pl.ds(base, _CH)], idx_v)
        pltpu.async_copy(att2.at[idx_v], rows_v, sem).wait()
        pltpu.sync_copy(rows_v, att_out.at[pl.ds(base, _CH)])
        return carry

    lax.fori_loop(0, _ATT_CHUNKS, chunk, 0)


def _sc_rep_body(g_hbm, rep2, rep_out, idx_v, rows_v, sem):
    wid = lax.axis_index("s") * _NC + lax.axis_index("c")
    base0 = wid * _REP_PER_W

    def chunk(j, carry):
        base = base0 + j * _CH
        pltpu.sync_copy(g_hbm.at[pl.ds(base, _CH)], idx_v)
        pltpu.async_copy(rep2.at[idx_v], rows_v, sem).wait()
        pltpu.sync_copy(rows_v, rep_out.at[pl.ds(base, _CH)])
        return carry

    lax.fori_loop(0, _REP_CHUNKS, chunk, 0)


_BAT = 16


def _sc_cat_body(catt_hbm, cat_tbl, cat_out, idx_v, rows_v, sem):
    wid = lax.axis_index("s") * _NC + lax.axis_index("c")
    base0 = wid * _CAT_PER_W

    def chunk(j, carry):
        base = base0 + j * _CH
        field = base // _B
        pltpu.sync_copy(catt_hbm.at[pl.ds(base, _CH)], idx_v)

        def batch(jj, c2):
            i0 = jj * _BAT
            rv = idx_v[pl.ds(i0, _BAT)]
            for t in range(_BAT):
                pltpu.async_copy(cat_tbl.at[field, rv[t]],
                                 rows_v.at[i0 + t], sem)
            for t in range(_BAT):
                pltpu.make_async_copy(cat_tbl.at[field, rv[t]],
                                      rows_v.at[i0 + t], sem).wait()
            return c2

        lax.fori_loop(0, _CH // _BAT, batch, 0)
        pltpu.sync_copy(rows_v, cat_out.at[pl.ds(base, _CH)])
        return carry

    lax.fori_loop(0, _CAT_CHUNKS, chunk, 0)


@functools.lru_cache(maxsize=None)
def _sc_kernels():
    mesh = plsc.VectorSubcoreMesh(core_axis_name="c", subcore_axis_name="s",
                                  num_cores=_NC, num_subcores=_NS)
    wide = [
        pltpu.VMEM((_CH,), jnp.int32),
        pltpu.VMEM((_CH, 2 * _D), jnp.float32),
        pltpu.SemaphoreType.DMA,
    ]
    narrow = [
        pltpu.VMEM((_CH,), jnp.int32),
        pltpu.VMEM((_CH, _D), jnp.float32),
        pltpu.SemaphoreType.DMA,
    ]
    katt = pl.kernel(
        _sc_att_body,
        out_type=jax.ShapeDtypeStruct((_ATT_ROWS, 2 * _D), jnp.float32),
        mesh=mesh,
        scratch_types=wide,
    )
    kcat = pl.kernel(
        _sc_cat_body,
        out_type=jax.ShapeDtypeStruct((_CAT_ROWS, _D), jnp.float32),
        mesh=mesh,
        scratch_types=narrow,
    )
    krep = pl.kernel(
        _sc_rep_body,
        out_type=jax.ShapeDtypeStruct((_REP_ROWS, 2 * _D), jnp.float32),
        mesh=mesh,
        scratch_types=wide,
    )
    return katt, kcat, krep


_BB1 = 128


def _tc1_body(att_ref, cat_ref, seq_ref, xnum_ref, xmask_ref, nw1_ref, np_ref,
              mw1_ref, mp_ref, cproj_ref, fc1r_ref, fc1b_ref,
              part_ref, w_ref, g_ref):
    f32 = jnp.float32
    cats = []
    for f in range(_FC):
        ce = jnp.dot(cat_ref[f], cproj_ref[f], preferred_element_type=f32)
        cats.append(ce)
    q = cats[0]

    nm = jnp.dot(jnp.dot(xnum_ref[...], nw1_ref[...],
                         preferred_element_type=f32), np_ref[...],
                 preferred_element_type=f32) * (1.0 / 13.0)
    mm = jnp.dot(jnp.dot(xmask_ref[...], mw1_ref[...],
                         preferred_element_type=f32), mp_ref[...],
                 preferred_element_type=f32) * (1.0 / 13.0)

    att = att_ref[...]
    qext = jnp.concatenate([q, jnp.zeros((_BB1, _D), f32)], axis=1)
    s = jnp.sum(att * qext[:, None, :], axis=-1) * 0.125
    iota_l = lax.broadcasted_iota(jnp.int32, (_BB1, _L), 1)
    bias = (jnp.float32(_L - 1) - iota_l.astype(f32)) * (1.0 / 50.0)
    s = s - bias
    seqb = seq_ref[...]
    s = jnp.where(seqb == 0, jnp.float32(-1e9), s)

    vals, gsel = [], []
    for _ in range(_K):
        m = jnp.max(s, axis=1, keepdims=True)
        cand = jnp.where(s == m, iota_l, jnp.int32(_L))
        ji = jnp.min(cand, axis=1, keepdims=True)
        onehot = iota_l == ji
        gk = jnp.sum(jnp.where(onehot, seqb, 0), axis=1, keepdims=True)
        s = jnp.where(onehot, jnp.float32(-3e38), s)
        vals.append(m)
        gsel.append(gk)
    vals = jnp.concatenate(vals, axis=1)
    gsel = jnp.concatenate(gsel, axis=1)
    vmax = jnp.max(vals, axis=1, keepdims=True)
    e = jnp.exp(vals - vmax)
    w = e / jnp.sum(e, axis=1, keepdims=True)

    feats = jnp.concatenate([nm, mm] + cats, axis=1)
    part = jnp.dot(feats, fc1r_ref[...], preferred_element_type=f32)
    part = part + fc1b_ref[...]

    part_ref[...] = part
    w_ref[...] = w
    g_ref[...] = gsel


_BB2 = 256


def _tc2_body(rep_ref, w_ref, part_ref, fc1u_ref, auxw_ref, auxb_ref,
              fc2w_ref, fc2b_ref, logit_ref, prob_ref, aux_ref):
    f32 = jnp.float32
    w = w_ref[...]
    u = jnp.zeros((_BB2, _D), f32)
    for k in range(_K):
        u = u + w[:, k:k + 1] * rep_ref[:, k * 2 * _D:k * 2 * _D + _D]
    aux = jnp.dot(u, auxw_ref[...], preferred_element_type=f32) + auxb_ref[...]
    h = jnp.maximum(part_ref[...] +
                    jnp.dot(u, fc1u_ref[...], preferred_element_type=f32), 0.0)
    logit = jnp.dot(h, fc2w_ref[...], preferred_element_type=f32) + fc2b_ref[...]
    logit_ref[...] = logit
    prob_ref[...] = jax.nn.sigmoid(logit)
    aux_ref[...] = aux


def _tc1(att_g, cat_raw, seq, xnum, xmask, nw1, npj, mw1, mpj, cproj,
         fc1r, fc1b):
    grid = (_B // _BB1,)
    return pl.pallas_call(
        _tc1_body,
        grid=grid,
        in_specs=[
            pl.BlockSpec((_BB1, _L, 2 * _D), lambda i: (i, 0, 0)),
            pl.BlockSpec((_FC, _BB1, _D), lambda i: (0, i, 0)),
            pl.BlockSpec((_BB1, _L), lambda i: (i, 0)),
            pl.BlockSpec((_BB1, 13), lambda i: (i, 0)),
            pl.BlockSpec((_BB1, 13), lambda i: (i, 0)),
            pl.BlockSpec((13, 16), lambda i: (0, 0)),
            pl.BlockSpec((16, _D), lambda i: (0, 0)),
            pl.BlockSpec((13, 16), lambda i: (0, 0)),
            pl.BlockSpec((16, _D), lambda i: (0, 0)),
            pl.BlockSpec((_FC, _D, _D), lambda i: (0, 0, 0)),
            pl.BlockSpec((28 * _D, 512), lambda i: (0, 0)),
            pl.BlockSpec((1, 512), lambda i: (0, 0)),
        ],
        out_specs=[
            pl.BlockSpec((_BB1, 512), lambda i: (i, 0)),
            pl.BlockSpec((_BB1, _K), lambda i: (i, 0)),
            pl.BlockSpec((_BB1, _K), lambda i: (i, 0)),
        ],
        out_shape=[
            jax.ShapeDtypeStruct((_B, 512), jnp.float32),
            jax.ShapeDtypeStruct((_B, _K), jnp.float32),
            jax.ShapeDtypeStruct((_B, _K), jnp.int32),
        ],
    )(att_g, cat_raw, seq, xnum, xmask, nw1, npj, mw1, mpj, cproj, fc1r, fc1b)


def _tc2(rep2, w, part, fc1u, auxw, auxb, fc2w, fc2b):
    grid = (_B // _BB2,)
    return pl.pallas_call(
        _tc2_body,
        grid=grid,
        in_specs=[
            pl.BlockSpec((_BB2, _K * 2 * _D), lambda i: (i, 0)),
            pl.BlockSpec((_BB2, _K), lambda i: (i, 0)),
            pl.BlockSpec((_BB2, 512), lambda i: (i, 0)),
            pl.BlockSpec((_D, 512), lambda i: (0, 0)),
            pl.BlockSpec((_D, 1), lambda i: (0, 0)),
            pl.BlockSpec((1, 1), lambda i: (0, 0)),
            pl.BlockSpec((512, 1), lambda i: (0, 0)),
            pl.BlockSpec((1, 1), lambda i: (0, 0)),
        ],
        out_specs=[
            pl.BlockSpec((_BB2, 1), lambda i: (i, 0)),
            pl.BlockSpec((_BB2, 1), lambda i: (i, 0)),
            pl.BlockSpec((_BB2, 1), lambda i: (i, 0)),
        ],
        out_shape=[
            jax.ShapeDtypeStruct((_B, 1), jnp.float32),
            jax.ShapeDtypeStruct((_B, 1), jnp.float32),
            jax.ShapeDtypeStruct((_B, 1), jnp.float32),
        ],
    )(rep2, w, part, fc1u, auxw, auxb, fc2w, fc2b)


def kernel(X_num, X_mask, X_cat, seq, num_W1, num_P, mask_W1, mask_P,
           cat_tables, cat_proj, att_table, rep_table, aux_w, aux_b,
           fc1_w, fc1_b, fc2_w, fc2_b):
    seq = seq.astype(jnp.int32)
    X_cat = X_cat.astype(jnp.int32)
    seq_flat = seq.reshape(-1)
    catt = X_cat.T.reshape(-1)

    katt, kcat, krep = _sc_kernels()

    cat_raw = kcat(catt, cat_tables)
    att2 = _tx(att_table)
    att_g = katt(seq_flat, att2)

    part, w, g = _tc1(
        att_g.reshape(_B, _L, 2 * _D), cat_raw.reshape(_FC, _B, _D), seq,
        X_num, X_mask, num_W1, num_P, mask_W1, mask_P, cat_proj,
        fc1_w[_D:], fc1_b.reshape(1, 512))

    rep2 = _tx(rep_table)
    rep_g = krep(g.reshape(-1), rep2)

    logit, prob, aux = _tc2(
        rep_g.reshape(_B, _K * 2 * _D), w, part, fc1_w[:_D],
        aux_w.reshape(_D, 1), aux_b.reshape(1, 1), fc2_w,
        fc2_b.reshape(1, 1))

    return (logit[:, 0], prob[:, 0], aux[:, 0])
